# baseline (speedup 1.0000x reference)
import functools

import jax
import jax.numpy as jnp
from jax import lax
from jax.experimental import pallas as pl
from jax.experimental.pallas import tpu as pltpu
from jax.experimental.pallas import tpu_sc as plsc

N = 100000
E = 3200000
L = 16
NC, NS = 2, 16
NW = NC * NS
NP = 100352
ROWS = 25600
RPW = ROWS // NW
G = 8
NCHUNK = RPW // G
PAD_IDX = N

_mesh = plsc.VectorSubcoreMesh(
    core_axis_name="c", subcore_axis_name="s", num_cores=NC, num_subcores=NS
)


def _make_edge_pass(gather: bool):
    D = 4
    scratch = [
        pltpu.VMEM_SHARED((NP,), jnp.float32),
        pltpu.VMEM((D, G, 128), jnp.int32),
        pltpu.VMEM((D, G, 128), jnp.float32),
        [pltpu.SemaphoreType.DMA] * D,
        [pltpu.SemaphoreType.DMA] * D,
    ]
    if gather:
        scratch.append(pltpu.VMEM((NP,), jnp.float32))
        scratch.append(pltpu.VMEM((D, G, 128), jnp.int32))

    def body(*refs):
        if gather:
            (q_hbm, src_hbm, dst_hbm, zero_hbm, out_hbm,
             acc_sh, dst_v, vals_v, lsem, ssem, q_v, src_v) = refs
        else:
            (dst_hbm, zero_hbm, out_hbm,
             acc_sh, dst_v, vals_v, lsem, ssem) = refs

        cid = lax.axis_index("c")
        sid = lax.axis_index("s")
        RPW_A, RPW_B = 1280, 320
        base = sid * (RPW_A + RPW_B) + cid * RPW_A
        n_chunk = jnp.where(cid == 0, RPW_A // G, RPW_B // G)

        @pl.when(sid == 0)
        def _zero():
            pltpu.sync_copy(zero_hbm, acc_sh)

        def fire_load(k, b):
            r0 = jnp.minimum(base + k * G, ROWS - G)
            pltpu.async_copy(dst_hbm.at[pl.ds(r0, G)], dst_v.at[b], lsem[b])
            if gather:
                pltpu.async_copy(src_hbm.at[pl.ds(r0, G)], src_v.at[b], lsem[b])

        def wait_load(b):
            pltpu.make_async_copy(dst_hbm.at[pl.ds(0, G)], dst_v.at[b],
                                  lsem[b]).wait()
            if gather:
                pltpu.make_async_copy(src_hbm.at[pl.ds(0, G)], src_v.at[b],
                                      lsem[b]).wait()

        def fire_scatter(b):
            for j in range(G):
                pltpu.async_copy(vals_v.at[b].at[j],
                                 acc_sh.at[dst_v.at[b].at[j]],
                                 ssem[b], add=True)

        def drain_scatter(b):
            for j in range(G):
                pltpu.make_async_copy(vals_v.at[b].at[j],
                                      acc_sh.at[dst_v.at[b].at[j]],
                                      ssem[b]).wait()

        if gather:
            pltpu.sync_copy(q_hbm, q_v)
        else:
            ones = jnp.full((L,), 1.0, dtype=jnp.float32)
            for b in range(D):
                for j in range(G):
                    for c in range(128 // L):
                        vals_v[b, j, pl.ds(c * L, L)] = ones

        plsc.subcore_barrier()

        fire_load(0, 0)
        fire_load(1, 1)

        def chunk_body(m, carry):
            for b in range(D):
                k = m * D + b
                wait_load(b)
                if gather:
                    for j in range(G):
                        for c in range(128 // L):
                            idx = src_v[b, j, pl.ds(c * L, L)]
                            vals_v[b, j, pl.ds(c * L, L)] = plsc.load_gather(
                                q_v, [idx])
                fire_scatter(b)
                b2 = (b + 2) % D
                @pl.when(k >= 2)
                def _():
                    drain_scatter(b2)
                fire_load(k + 2, b2)
            return carry

        lax.fori_loop(0, n_chunk // D, chunk_body, 0)

        drain_scatter(2)
        drain_scatter(3)
        wait_load(0)
        wait_load(1)

        plsc.subcore_barrier()

        @pl.when(sid == 0)
        def _writeout():
            pltpu.sync_copy(acc_sh, out_hbm.at[cid])

    return pl.kernel(
        body,
        out_type=jax.ShapeDtypeStruct((NC, NP), jnp.float32),
        mesh=_mesh,
        scratch_types=scratch,
        compiler_params=pltpu.CompilerParams(needs_layout_passes=False),
    )


_deg_pass = _make_edge_pass(gather=False)
_msg_pass = _make_edge_pass(gather=True)


def _stage1_body(degp_ref, xt_ref, w1_ref, w2_ref, dinv_ref, q1_ref, p_ref):
    deg = degp_ref[0] + degp_ref[1] + 1.0
    dinv = lax.rsqrt(deg)
    w2col = w2_ref[:, 0]
    p = jnp.zeros((NP,), jnp.float32)
    for k in range(6):
        wk = jnp.sum(w1_ref[k, :] * w2col)
        p = p + xt_ref[k, :] * wk
    dinv_ref[...] = dinv
    p_ref[...] = p
    q1_ref[...] = dinv * p


def _stage2_body(s1p_ref, dinv_ref, p_ref, b1_ref, w2_ref, h2_ref, q2_ref):
    dinv = dinv_ref[...]
    c1 = jnp.sum(b1_ref[...] * w2_ref[:, 0])
    h2 = dinv * (s1p_ref[0] + s1p_ref[1]) + dinv * dinv * p_ref[...] + c1
    h2_ref[...] = h2
    q2_ref[...] = dinv * h2


def _stage3_body(s2p_ref, dinv_ref, h2_ref, b2_ref, out_ref):
    dinv = dinv_ref[...]
    o = dinv * (s2p_ref[0] + s2p_ref[1]) + dinv * dinv * h2_ref[...] + b2_ref[0]
    out_ref[...] = jnp.maximum(o, 0.01 * o)


_stage1 = pl.pallas_call(
    _stage1_body,
    out_shape=[jax.ShapeDtypeStruct((NP,), jnp.float32)] * 3,
)
_stage2 = pl.pallas_call(
    _stage2_body,
    out_shape=[jax.ShapeDtypeStruct((NP,), jnp.float32)] * 2,
)
_stage3 = pl.pallas_call(
    _stage3_body,
    out_shape=jax.ShapeDtypeStruct((NP,), jnp.float32),
)


@jax.jit
def kernel(x, edge_index, W1, b1, W2, b2):
    pad_e = ROWS * 128 - E
    src = jnp.concatenate(
        [edge_index[0], jnp.full((pad_e,), PAD_IDX, jnp.int32)]
    ).reshape(ROWS, 128)
    dst = jnp.concatenate(
        [edge_index[1], jnp.full((pad_e,), PAD_IDX, jnp.int32)]
    ).reshape(ROWS, 128)
    x_t = jnp.zeros((6, NP), jnp.float32).at[:, :N].set(x.T)
    zero = jnp.zeros((NP,), jnp.float32)

    degp = _deg_pass(dst, zero)
    dinv, q1, p = _stage1(degp, x_t, W1, W2)
    s1p = _msg_pass(q1, src, dst, zero)
    h2, q2 = _stage2(s1p, dinv, p, b1, W2)
    s2p = _msg_pass(q2, src, dst, zero)
    out = _stage3(s2p, dinv, h2, b2)
    return out[:N]


# speedup vs baseline: 1.1626x; 1.1626x over previous
import functools

import jax
import jax.numpy as jnp
from jax import lax
from jax.experimental import pallas as pl
from jax.experimental.pallas import tpu as pltpu
from jax.experimental.pallas import tpu_sc as plsc

N = 100000
E = 3200000
L = 16
NC, NS = 2, 16
NW = NC * NS
NP = 100352
ROWS = 25600
RPW = ROWS // NW
G = 8
NCHUNK = RPW // G
PAD_IDX = N

_mesh = plsc.VectorSubcoreMesh(
    core_axis_name="c", subcore_axis_name="s", num_cores=NC, num_subcores=NS
)


def _make_edge_pass(gather: bool):
    D = 4
    scratch = [
        pltpu.VMEM_SHARED((NP,), jnp.float32),
        pltpu.VMEM((D, G, 128), jnp.int32),
        pltpu.VMEM((D, G, 128), jnp.float32),
        [pltpu.SemaphoreType.DMA] * D,
        [pltpu.SemaphoreType.DMA] * D,
    ]
    if gather:
        scratch.append(pltpu.VMEM((NP,), jnp.float32))
        scratch.append(pltpu.VMEM((D, G, 128), jnp.int32))

    def body(*refs):
        if gather:
            (q_hbm, src_hbm, dst_hbm, zero_hbm, out_hbm,
             acc_sh, dst_v, vals_v, lsem, ssem, q_v, src_v) = refs
        else:
            (dst_hbm, zero_hbm, out_hbm,
             acc_sh, dst_v, vals_v, lsem, ssem) = refs

        cid = lax.axis_index("c")
        sid = lax.axis_index("s")
        RPW_A, RPW_B = 1120, 480
        base = sid * (RPW_A + RPW_B) + cid * RPW_A
        n_chunk = jnp.where(cid == 0, RPW_A // G, RPW_B // G)

        @pl.when(sid == 0)
        def _zero():
            pltpu.sync_copy(zero_hbm, acc_sh)

        def fire_load(k, b):
            r0 = jnp.minimum(base + k * G, ROWS - G)
            pltpu.async_copy(dst_hbm.at[pl.ds(r0, G)], dst_v.at[b], lsem[b])
            if gather:
                pltpu.async_copy(src_hbm.at[pl.ds(r0, G)], src_v.at[b], lsem[b])

        def wait_load(b):
            pltpu.make_async_copy(dst_hbm.at[pl.ds(0, G)], dst_v.at[b],
                                  lsem[b]).wait()
            if gather:
                pltpu.make_async_copy(src_hbm.at[pl.ds(0, G)], src_v.at[b],
                                      lsem[b]).wait()

        def fire_scatter(b):
            for j in range(G):
                pltpu.async_copy(vals_v.at[b].at[j],
                                 acc_sh.at[dst_v.at[b].at[j]],
                                 ssem[b], add=True)

        def drain_scatter(b):
            for j in range(G):
                pltpu.make_async_copy(vals_v.at[b].at[j],
                                      acc_sh.at[dst_v.at[b].at[j]],
                                      ssem[b]).wait()

        if gather:
            pltpu.sync_copy(q_hbm, q_v)
        else:
            ones = jnp.full((L,), 1.0, dtype=jnp.float32)
            for b in range(D):
                for j in range(G):
                    for c in range(128 // L):
                        vals_v[b, j, pl.ds(c * L, L)] = ones

        plsc.subcore_barrier()

        fire_load(0, 0)
        fire_load(1, 1)

        def chunk_body(m, carry):
            for b in range(D):
                k = m * D + b
                wait_load(b)
                if gather:
                    for j in range(G):
                        for c in range(128 // L):
                            idx = src_v[b, j, pl.ds(c * L, L)]
                            vals_v[b, j, pl.ds(c * L, L)] = plsc.load_gather(
                                q_v, [idx])
                fire_scatter(b)
                b2 = (b + 2) % D
                @pl.when(k >= 2)
                def _():
                    drain_scatter(b2)
                fire_load(k + 2, b2)
            return carry

        lax.fori_loop(0, n_chunk // D, chunk_body, 0)

        drain_scatter(2)
        drain_scatter(3)
        wait_load(0)
        wait_load(1)

        plsc.subcore_barrier()

        @pl.when(sid == 0)
        def _writeout():
            pltpu.sync_copy(acc_sh, out_hbm.at[cid])

    return pl.kernel(
        body,
        out_type=jax.ShapeDtypeStruct((NC, NP), jnp.float32),
        mesh=_mesh,
        scratch_types=scratch,
        compiler_params=pltpu.CompilerParams(needs_layout_passes=False),
    )


_deg_pass = _make_edge_pass(gather=False)
_msg_pass = _make_edge_pass(gather=True)


def _stage1_body(degp_ref, xt_ref, w1_ref, w2_ref, dinv_ref, q1_ref, p_ref):
    deg = degp_ref[0] + degp_ref[1] + 1.0
    dinv = lax.rsqrt(deg)
    w2col = w2_ref[:, 0]
    p = jnp.zeros((NP,), jnp.float32)
    for k in range(6):
        wk = jnp.sum(w1_ref[k, :] * w2col)
        p = p + xt_ref[k, :] * wk
    dinv_ref[...] = dinv
    p_ref[...] = p
    q1_ref[...] = dinv * p


def _stage2_body(s1p_ref, dinv_ref, p_ref, b1_ref, w2_ref, h2_ref, q2_ref):
    dinv = dinv_ref[...]
    c1 = jnp.sum(b1_ref[...] * w2_ref[:, 0])
    h2 = dinv * (s1p_ref[0] + s1p_ref[1]) + dinv * dinv * p_ref[...] + c1
    h2_ref[...] = h2
    q2_ref[...] = dinv * h2


def _stage3_body(s2p_ref, dinv_ref, h2_ref, b2_ref, out_ref):
    dinv = dinv_ref[...]
    o = dinv * (s2p_ref[0] + s2p_ref[1]) + dinv * dinv * h2_ref[...] + b2_ref[0]
    out_ref[...] = jnp.maximum(o, 0.01 * o)


_stage1 = pl.pallas_call(
    _stage1_body,
    out_shape=[jax.ShapeDtypeStruct((NP,), jnp.float32)] * 3,
)
_stage2 = pl.pallas_call(
    _stage2_body,
    out_shape=[jax.ShapeDtypeStruct((NP,), jnp.float32)] * 2,
)
_stage3 = pl.pallas_call(
    _stage3_body,
    out_shape=jax.ShapeDtypeStruct((NP,), jnp.float32),
)


@jax.jit
def kernel(x, edge_index, W1, b1, W2, b2):
    pad_e = ROWS * 128 - E
    src = jnp.concatenate(
        [edge_index[0], jnp.full((pad_e,), PAD_IDX, jnp.int32)]
    ).reshape(ROWS, 128)
    dst = jnp.concatenate(
        [edge_index[1], jnp.full((pad_e,), PAD_IDX, jnp.int32)]
    ).reshape(ROWS, 128)
    x_t = jnp.zeros((6, NP), jnp.float32).at[:, :N].set(x.T)
    zero = jnp.zeros((NP,), jnp.float32)

    degp = _deg_pass(dst, zero)
    dinv, q1, p = _stage1(degp, x_t, W1, W2)
    s1p = _msg_pass(q1, src, dst, zero)
    h2, q2 = _stage2(s1p, dinv, p, b1, W2)
    s2p = _msg_pass(q2, src, dst, zero)
    out = _stage3(s2p, dinv, h2, b2)
    return out[:N]
